# Initial kernel scaffold; baseline (speedup 1.0000x reference)
import jax
import jax.numpy as jnp
from jax.experimental import pallas as pl


def kernel(x, edge_index, gamma, beta, Wp, bp, Wl, bl, Wr):
    raise NotImplementedError("write your pallas kernel here")


# speedup vs baseline: 5.0250x; 5.0250x over previous
import functools

import jax
import jax.numpy as jnp
from jax import lax
from jax.experimental import pallas as pl
from jax.experimental.pallas import tpu as pltpu
from jax.experimental.pallas import tpu_sc as plsc

N = 10000
E = 320000
D = 128

NW = 32
CHUNK = 128
CHUNKS_PER_W = 79
E_PAD = NW * CHUNKS_PER_W * CHUNK
N_PAD = 10240
ROWS_PER_TILE = N_PAD // 16

DEG_A = N_PAD // D
EBLK = 8000


def _ln_proj_body(x_ref, g_ref, b_ref, wpt_ref, bp_ref, h_ref, hp_ref):
    x = x_ref[...]
    mu = jnp.mean(x, axis=1, keepdims=True)
    var = jnp.mean((x - mu) ** 2, axis=1, keepdims=True)
    h = (x - mu) * lax.rsqrt(var + 1e-5)
    h = h * g_ref[...] + b_ref[...]
    h = jnp.maximum(h, 0.0)
    h_ref[...] = h
    hp = jnp.dot(h, wpt_ref[...], preferred_element_type=jnp.float32) + bp_ref[...]
    hp_ref[...] = jnp.maximum(hp, 0.0)


def _ln_proj(x, gamma, beta, WpT, bp):
    blk = 1000
    grid = N // blk
    return pl.pallas_call(
        _ln_proj_body,
        grid=(grid,),
        in_specs=[
            pl.BlockSpec((blk, D), lambda i: (i, 0)),
            pl.BlockSpec((1, D), lambda i: (0, 0)),
            pl.BlockSpec((1, D), lambda i: (0, 0)),
            pl.BlockSpec((D, D), lambda i: (0, 0)),
            pl.BlockSpec((1, D), lambda i: (0, 0)),
        ],
        out_specs=[
            pl.BlockSpec((blk, D), lambda i: (i, 0)),
            pl.BlockSpec((blk, D), lambda i: (i, 0)),
        ],
        out_shape=[
            jax.ShapeDtypeStruct((N, D), jnp.float32),
            jax.ShapeDtypeStruct((N, D), jnp.float32),
        ],
    )(x, gamma, beta, WpT, bp)


def _deg_body(dst_ref, deg_ref):
    step = pl.program_id(0)
    d = dst_ref[...]
    a = jnp.right_shift(d, 7)
    b = jnp.bitwise_and(d, 127)
    a_oh = (a == lax.broadcasted_iota(jnp.int32, (EBLK, DEG_A), 1)
            ).astype(jnp.bfloat16)
    b_oh = (b == lax.broadcasted_iota(jnp.int32, (EBLK, D), 1)
            ).astype(jnp.bfloat16)
    partial = lax.dot_general(a_oh, b_oh, (((0,), (0,)), ((), ())),
                              preferred_element_type=jnp.float32)

    @pl.when(step == 0)
    def _():
        deg_ref[...] = jnp.zeros_like(deg_ref)

    deg_ref[...] += partial


def _deg_histogram(dst_col):
    grid = E // EBLK
    return pl.pallas_call(
        _deg_body,
        grid=(grid,),
        in_specs=[pl.BlockSpec((EBLK, 1), lambda i: (i, 0))],
        out_specs=pl.BlockSpec((DEG_A, D), lambda i: (0, 0)),
        out_shape=jax.ShapeDtypeStruct((DEG_A, D), jnp.float32),
    )(dst_col)


def _sc_segment_sum_body(src_hbm, dst_hbm, hproj_hbm, zfeat_hbm,
                         sums_out, src_v, dst_v, rows_v, sum_sh, sem):
    cid = lax.axis_index("c")
    sid = lax.axis_index("s")
    wid = cid * 16 + sid

    r0 = sid * ROWS_PER_TILE
    pltpu.sync_copy(zfeat_hbm.at[pl.ds(r0, ROWS_PER_TILE)],
                    sum_sh.at[pl.ds(r0, ROWS_PER_TILE)])

    pltpu.sync_copy(src_hbm.at[wid], src_v)
    pltpu.sync_copy(dst_hbm.at[wid], dst_v)

    plsc.subcore_barrier()

    def body(j, carry):
        pltpu.async_copy(hproj_hbm.at[src_v.at[j]], rows_v, sem).wait()
        pltpu.sync_copy(rows_v, sum_sh.at[dst_v.at[j]], add=True)
        return carry

    lax.fori_loop(0, CHUNKS_PER_W, body, 0)

    plsc.subcore_barrier()

    o0 = cid * N_PAD + r0
    pltpu.sync_copy(sum_sh.at[pl.ds(r0, ROWS_PER_TILE)],
                    sums_out.at[pl.ds(o0, ROWS_PER_TILE)])


_sc_segment_sum = functools.partial(
    pl.kernel,
    out_type=jax.ShapeDtypeStruct((2 * N_PAD, D), jnp.float32),
    mesh=plsc.VectorSubcoreMesh(core_axis_name="c", subcore_axis_name="s"),
    scratch_types=[
        pltpu.VMEM((CHUNKS_PER_W, CHUNK), jnp.int32),
        pltpu.VMEM((CHUNKS_PER_W, CHUNK), jnp.int32),
        pltpu.VMEM((CHUNK, D), jnp.float32),
        pltpu.VMEM_SHARED((N_PAD, D), jnp.float32),
        pltpu.SemaphoreType.DMA,
    ],
)(_sc_segment_sum_body)


def _out_body(h_ref, sums_ref, degs_ref, wlt_ref, bl_ref, wrt_ref, o_ref):
    s = sums_ref[...]
    summed = s[0] + s[1]
    deg = degs_ref[...]
    aggr = summed / jnp.maximum(deg, 1.0)
    o_ref[...] = (jnp.dot(aggr, wlt_ref[...], preferred_element_type=jnp.float32)
                  + bl_ref[...]
                  + jnp.dot(h_ref[...], wrt_ref[...],
                            preferred_element_type=jnp.float32))


def _combine(h, sums, degs, WlT, bl, WrT):
    blk = 1000
    grid = N // blk
    return pl.pallas_call(
        _out_body,
        grid=(grid,),
        in_specs=[
            pl.BlockSpec((blk, D), lambda i: (i, 0)),
            pl.BlockSpec((2, blk, D), lambda i: (0, i, 0)),
            pl.BlockSpec((blk, 1), lambda i: (i, 0)),
            pl.BlockSpec((D, D), lambda i: (0, 0)),
            pl.BlockSpec((1, D), lambda i: (0, 0)),
            pl.BlockSpec((D, D), lambda i: (0, 0)),
        ],
        out_specs=pl.BlockSpec((blk, D), lambda i: (i, 0)),
        out_shape=jax.ShapeDtypeStruct((N, D), jnp.float32),
    )(h, sums, degs, WlT, bl, WrT)


def kernel(x, edge_index, gamma, beta, Wp, bp, Wl, bl, Wr):
    ei = edge_index.astype(jnp.int32)
    pad = E_PAD - E
    src = jnp.concatenate([ei[0], jnp.zeros((pad,), jnp.int32)])
    dst = jnp.concatenate([ei[1], jnp.full((pad,), N_PAD - 1, jnp.int32)])
    src = src.reshape(NW, CHUNKS_PER_W, CHUNK)
    dst = dst.reshape(NW, CHUNKS_PER_W, CHUNK)

    h, hproj = _ln_proj(x, gamma.reshape(1, D), beta.reshape(1, D),
                        Wp.T, bp.reshape(1, D))
    deg = _deg_histogram(ei[1].reshape(E, 1))

    zfeat = jnp.zeros((N_PAD, D), jnp.float32)
    sums = _sc_segment_sum(src, dst, hproj, zfeat)

    return _combine(h, sums.reshape(2, N_PAD, D), deg.reshape(N_PAD, 1),
                    Wl.T, bl.reshape(1, D), Wr.T)
